# baseline (speedup 1.0000x reference)
import functools

import jax
import jax.numpy as jnp
from jax.experimental import pallas as pl
from jax.experimental.pallas import tpu as pltpu

BUCKET_SIZE = 64
N_HASHES = 8
TOKEN_SELF_ATTN_VALUE = -10000.0
SEQLEN = 4096
BATCH = 8
DIM = 64
N_BUCKETS = SEQLEN // BUCKET_SIZE
N_CHUNKS = N_HASHES * N_BUCKETS
NBINS = N_HASHES * N_BUCKETS
BLK = 128
NBLK = SEQLEN // BLK

_INTERPRET = False


def _hash_rank_kernel(qk_ref, rot_ref, rank_ref, sb_ref, oh_ref, win_ref):
    qk = qk_ref[0]
    rot = rot_ref[...]
    rotated = jnp.dot(qk, rot)

    cols = []
    iota32 = jax.lax.broadcasted_iota(jnp.int32, (SEQLEN, 32), 1)
    for h in range(N_HASHES):
        r = rotated[:, h * 32:(h + 1) * 32]
        m1 = jnp.max(r, axis=-1, keepdims=True)
        m2 = jnp.max(-r, axis=-1, keepdims=True)
        m = jnp.maximum(m1, m2)
        big = jnp.int32(10_000)
        idx_pos = jnp.min(jnp.where(r == m, iota32, big), axis=-1, keepdims=True)
        idx_neg = jnp.min(jnp.where(-r == m, iota32 + 32, big), axis=-1, keepdims=True)
        bucket = jnp.minimum(idx_pos, idx_neg) + h * N_BUCKETS
        cols.append(bucket)
    bucket_glob = jnp.concatenate(cols, axis=1)

    bcast = jnp.concatenate(
        [jnp.broadcast_to(bucket_glob[:, h:h + 1], (SEQLEN, N_BUCKETS))
         for h in range(N_HASHES)], axis=1)
    iota512 = jax.lax.broadcasted_iota(jnp.int32, (1, NBINS), 1)
    oh = (bcast == iota512).astype(jnp.float32)
    oh_ref[...] = oh

    gcol = jax.lax.broadcasted_iota(jnp.int32, (NBINS, N_HASHES), 0) // N_BUCKETS
    ghdr = jax.lax.broadcasted_iota(jnp.int32, (NBINS, N_HASHES), 1)
    G = (gcol == ghdr).astype(jnp.float32)

    ii = jax.lax.broadcasted_iota(jnp.int32, (BLK, BLK), 0)
    jj = jax.lax.broadcasted_iota(jnp.int32, (BLK, BLK), 1)
    tril = (ii >= jj).astype(jnp.float32)

    def body(i, carry):
        blk = oh_ref[pl.ds(i * BLK, BLK), :]
        C = jnp.dot(tril, blk, preferred_element_type=jnp.float32) + carry
        win_ref[pl.ds(i * BLK, BLK), :] = jnp.dot(blk * C, G,
                                                  preferred_element_type=jnp.float32)
        return carry + jnp.sum(blk, axis=0, keepdims=True)

    hist = jax.lax.fori_loop(0, NBLK, body, jnp.zeros((1, NBINS), jnp.float32))

    bi = jax.lax.broadcasted_iota(jnp.int32, (NBINS, NBINS), 0)
    bj = jax.lax.broadcasted_iota(jnp.int32, (NBINS, NBINS), 1)
    same_grp = (bi // N_BUCKETS) == (bj // N_BUCKETS)
    strictU = (same_grp & (bi < bj)).astype(jnp.float32)
    off = jnp.dot(hist, strictU, preferred_element_type=jnp.float32)

    off_elem = jnp.dot(oh * off, G, preferred_element_type=jnp.float32)
    rank = win_ref[...] - 1.0 + off_elem
    rank_ref[0] = rank.astype(jnp.int32)

    pos = jax.lax.broadcasted_iota(jnp.int32, (SEQLEN, 1), 0).astype(jnp.float32)
    ge = (pos >= off).astype(jnp.float32)
    sb = jnp.dot(ge, G, preferred_element_type=jnp.float32) - 1.0
    hoff = (jax.lax.broadcasted_iota(jnp.int32, (1, N_HASHES), 1)
            * N_BUCKETS).astype(jnp.float32)
    sb_ref[0] = (sb + hoff).astype(jnp.int32)


def _hash_rank(qk, rot2):
    return pl.pallas_call(
        _hash_rank_kernel,
        grid=(BATCH,),
        in_specs=[
            pl.BlockSpec((1, SEQLEN, DIM), lambda b: (b, 0, 0)),
            pl.BlockSpec((DIM, N_HASHES * 32), lambda b: (0, 0)),
        ],
        out_specs=[
            pl.BlockSpec((1, SEQLEN, N_HASHES), lambda b: (b, 0, 0)),
            pl.BlockSpec((1, SEQLEN, N_HASHES), lambda b: (b, 0, 0)),
        ],
        out_shape=[
            jax.ShapeDtypeStruct((BATCH, SEQLEN, N_HASHES), jnp.int32),
            jax.ShapeDtypeStruct((BATCH, SEQLEN, N_HASHES), jnp.int32),
        ],
        scratch_shapes=[
            pltpu.VMEM((SEQLEN, NBINS), jnp.float32),
            pltpu.VMEM((SEQLEN, N_HASHES), jnp.float32),
        ],
        interpret=_INTERPRET,
    )(qk, rot2)


CH_PER = 16
NGRP = N_CHUNKS // CH_PER


def _attn_kernel(q_ref, v_ref, kprev_ref, vprev_ref, sbcol_ref, sbrow_ref,
                 sbrow_prev_ref, out_ref, lse_ref):
    fmax = jnp.finfo(jnp.float32).max
    ii = jax.lax.broadcasted_iota(jnp.int32, (BUCKET_SIZE, 2 * BUCKET_SIZE), 0)
    jj = jax.lax.broadcasted_iota(jnp.int32, (BUCKET_SIZE, 2 * BUCKET_SIZE), 1)
    diag = ii == jj

    for i in range(CH_PER):
        q = q_ref[0, pl.ds(i * BUCKET_SIZE, BUCKET_SIZE), :]
        if i == 0:
            kp = kprev_ref[0]
            vp = vprev_ref[0]
            sbk_p = sbrow_prev_ref[0, 0, 0:1, :]
        else:
            kp = q_ref[0, pl.ds((i - 1) * BUCKET_SIZE, BUCKET_SIZE), :]
            vp = v_ref[0, pl.ds((i - 1) * BUCKET_SIZE, BUCKET_SIZE), :]
            sbk_p = sbrow_ref[0, i - 1:i, 0, :]
        k = jnp.concatenate([q, kp], axis=0)
        norms = jnp.sqrt(jnp.sum(k * k, axis=-1, keepdims=True))
        kn = k / jnp.maximum(norms, 1e-12)
        dots = jax.lax.dot_general(q, kn, (((1,), (1,)), ((), ())),
                                   preferred_element_type=jnp.float32)
        dots = dots * (DIM ** -0.5)

        sbq = sbcol_ref[0, pl.ds(i * BUCKET_SIZE, BUCKET_SIZE), :]
        sbk_s = sbrow_ref[0, i:i + 1, 0, :]
        sbk = jnp.concatenate([sbk_s, sbk_p], axis=1)
        dots = jnp.where(diag, TOKEN_SELF_ATTN_VALUE, dots)
        dots = jnp.where(sbq != sbk, -fmax, dots)

        m = jnp.max(dots, axis=-1, keepdims=True)
        e = jnp.exp(dots - m)
        s = jnp.sum(e, axis=-1, keepdims=True)
        lse = m + jnp.log(s)
        probs = e / s
        vv = v_ref[0, pl.ds(i * BUCKET_SIZE, BUCKET_SIZE), :]
        bv = jnp.concatenate([vv, vp], axis=0)
        out_ref[0, pl.ds(i * BUCKET_SIZE, BUCKET_SIZE), :] = jnp.dot(
            probs, bv, preferred_element_type=jnp.float32)
        lse_ref[0, pl.ds(i * BUCKET_SIZE, BUCKET_SIZE), :] = lse


def _attention(sqk, sv, sb_col, sb_row):
    HS = N_HASHES * SEQLEN
    grid = (BATCH, NGRP)
    prev_blk = lambda b, g: (b, (g * CH_PER + N_CHUNKS - 1) % N_CHUNKS, 0)
    return pl.pallas_call(
        _attn_kernel,
        grid=grid,
        in_specs=[
            pl.BlockSpec((1, CH_PER * BUCKET_SIZE, DIM), lambda b, g: (b, g, 0)),
            pl.BlockSpec((1, CH_PER * BUCKET_SIZE, DIM), lambda b, g: (b, g, 0)),
            pl.BlockSpec((1, BUCKET_SIZE, DIM), prev_blk),
            pl.BlockSpec((1, BUCKET_SIZE, DIM), prev_blk),
            pl.BlockSpec((1, CH_PER * BUCKET_SIZE, 1), lambda b, g: (b, g, 0)),
            pl.BlockSpec((1, CH_PER, 1, BUCKET_SIZE), lambda b, g: (b, g, 0, 0)),
            pl.BlockSpec((1, 1, 1, BUCKET_SIZE),
                         lambda b, g: (b, (g * CH_PER + N_CHUNKS - 1) % N_CHUNKS, 0, 0)),
        ],
        out_specs=[
            pl.BlockSpec((1, CH_PER * BUCKET_SIZE, DIM), lambda b, g: (b, g, 0)),
            pl.BlockSpec((1, CH_PER * BUCKET_SIZE, 1), lambda b, g: (b, g, 0)),
        ],
        out_shape=[
            jax.ShapeDtypeStruct((BATCH, HS, DIM), jnp.float32),
            jax.ShapeDtypeStruct((BATCH, HS, 1), jnp.float32),
        ],
        interpret=_INTERPRET,
    )(sqk, sv, sqk, sv, sb_col, sb_row, sb_row)


POSB = 512


def _combine_kernel(o_ref, lg_ref, out_ref):
    lgs = [lg_ref[0, h] for h in range(N_HASHES)]
    m = lgs[0]
    for h in range(1, N_HASHES):
        m = jnp.maximum(m, lgs[h])
    es = [jnp.exp(lg - m) for lg in lgs]
    s = es[0]
    for h in range(1, N_HASHES):
        s = s + es[h]
    acc = jnp.zeros((POSB, DIM), jnp.float32)
    for h in range(N_HASHES):
        acc = acc + o_ref[0, h] * (es[h] / s)
    out_ref[0] = acc


def _combine(o, logits):
    return pl.pallas_call(
        _combine_kernel,
        grid=(BATCH, SEQLEN // POSB),
        in_specs=[
            pl.BlockSpec((1, N_HASHES, POSB, DIM), lambda b, p: (b, 0, p, 0)),
            pl.BlockSpec((1, N_HASHES, POSB, 1), lambda b, p: (b, 0, p, 0)),
        ],
        out_specs=pl.BlockSpec((1, POSB, DIM), lambda b, p: (b, p, 0)),
        out_shape=jax.ShapeDtypeStruct((BATCH, SEQLEN, DIM), jnp.float32),
        interpret=_INTERPRET,
    )(o, logits)


def kernel(qk, v):
    rot = jax.random.normal(jax.random.key(42), (DIM, N_HASHES, N_BUCKETS // 2),
                            dtype=qk.dtype)
    rot2 = rot.reshape(DIM, N_HASHES * (N_BUCKETS // 2))

    rank_t, sb_t = _hash_rank(qk, rot2)
    rank = jnp.transpose(rank_t, (0, 2, 1))
    sb = jnp.transpose(sb_t, (0, 2, 1))

    st_f = jnp.argsort(rank.reshape(BATCH * N_HASHES, SEQLEN), axis=-1)
    idx = st_f.reshape(BATCH, N_HASHES, SEQLEN)[..., None]
    sqk = jnp.take_along_axis(qk[:, None], idx, axis=2)
    sv = jnp.take_along_axis(v[:, None], idx, axis=2)
    sqk = sqk.reshape(BATCH, N_HASHES * SEQLEN, DIM)
    sv = sv.reshape(BATCH, N_HASHES * SEQLEN, DIM)

    sb_col = sb.reshape(BATCH, N_HASHES * SEQLEN, 1)
    sb_row = sb.reshape(BATCH, N_CHUNKS, 1, BUCKET_SIZE)

    so, lse = _attention(sqk, sv, sb_col, sb_row)

    so_r = so.reshape(BATCH, N_HASHES, SEQLEN, DIM)
    lse_r = lse.reshape(BATCH, N_HASHES, SEQLEN, 1)
    rank_idx = rank[..., None]
    o = jnp.take_along_axis(so_r, rank_idx, axis=2)
    logits = jnp.take_along_axis(lse_r, rank_idx, axis=2)

    return _combine(o, logits)
